# baseline (speedup 1.0000x reference)
import functools

import jax
import jax.numpy as jnp
from jax import lax
from jax.experimental import pallas as pl
from jax.experimental.pallas import tpu as pltpu
from jax.experimental.pallas import tpu_sc as plsc

NC = 2
NS = 16
NW = NC * NS
K = 1024
L = 16


@functools.lru_cache(maxsize=None)
def _build(B, TOTAL, NSEL_P):
    C = NSEL_P // K
    TASKS = B * C
    ITERS = (TASKS + NW - 1) // NW
    GRAN = TOTAL // L
    mesh = plsc.VectorSubcoreMesh(core_axis_name="c", subcore_axis_name="s")

    @functools.partial(
        pl.kernel,
        out_type=jax.ShapeDtypeStruct((B * NSEL_P,), jnp.float32),
        mesh=mesh,
        scratch_types=[
            pltpu.VMEM((K,), jnp.int32),
            pltpu.VMEM((K,), jnp.int32),
            pltpu.VMEM((K, L), jnp.float32),
            pltpu.VMEM((K,), jnp.float32),
            pltpu.SemaphoreType.DMA,
        ],
        compiler_params=pltpu.CompilerParams(
            needs_layout_passes=False, use_tc_tiling_on_sc=False
        ),
    )
    def gather_kernel(tab_hbm, idx_hbm, out_hbm, idx_v, row_v, rows_v,
                      val_v, sem):
        wid = lax.axis_index("s") * NC + lax.axis_index("c")
        lane_iota = lax.iota(jnp.int32, L)

        @pl.loop(0, ITERS)
        def _task(i):
            t = wid + i * NW

            @pl.when(t < TASKS)
            def _run():
                b = t % B
                c = t // B
                start = c * K
                pltpu.sync_copy(idx_hbm.at[pl.ds(start, K)], idx_v)

                row_base = b * GRAN

                @pl.loop(0, K // L)
                def _rows(j):
                    iv = idx_v[pl.ds(j * L, L)]
                    row_v[pl.ds(j * L, L)] = (
                        lax.shift_right_logical(iv, 4) + row_base
                    )

                pltpu.async_copy(tab_hbm.at[row_v], rows_v, sem).wait()

                @pl.loop(0, K // L)
                def _extract(j):
                    iv = idx_v[pl.ds(j * L, L)]
                    val_v[pl.ds(j * L, L)] = plsc.load_gather(
                        rows_v, [j * L + lane_iota, iv & (L - 1)]
                    )

                pltpu.sync_copy(
                    val_v, out_hbm.at[pl.ds(b * NSEL_P + start, K)]
                )

    return gather_kernel


def kernel(params, rp_cat, single_grad, flat_cat):
    B, TOTAL = params.shape
    NSEL = rp_cat.shape[0]
    if TOTAL == NSEL:
        return params
    NSEL_P = ((NSEL + K - 1) // K) * K
    rp = rp_cat.astype(jnp.int32)
    if NSEL_P != NSEL:
        rp = jnp.pad(rp, (0, NSEL_P - NSEL), mode="edge")
    out = _build(B, TOTAL, NSEL_P)(params.reshape(B * TOTAL // L, L), rp)
    return out.reshape(B, NSEL_P)[:, :NSEL]


# speedup vs baseline: 2.3829x; 2.3829x over previous
import functools

import jax
import jax.numpy as jnp
from jax import lax
from jax.experimental import pallas as pl
from jax.experimental.pallas import tpu as pltpu
from jax.experimental.pallas import tpu_sc as plsc

NC = 2
NS = 16
NW = NC * NS
K = 2048
L = 16


@functools.lru_cache(maxsize=None)
def _build(B, TOTAL, NSEL_P):
    C = NSEL_P // K
    G = B // 8
    TASKS = G * C
    ITERS = (TASKS + NW - 1) // NW
    HALF = TOTAL // 2
    WIN = HALF - 56
    mesh = plsc.VectorSubcoreMesh(core_axis_name="c", subcore_axis_name="s")

    @functools.partial(
        pl.kernel,
        out_type=jax.ShapeDtypeStruct((B * NSEL_P,), jnp.float32),
        mesh=mesh,
        scratch_types=[
            pltpu.VMEM((K,), jnp.int32),
            pltpu.VMEM((K,), jnp.int32),
            pltpu.VMEM((K,), jnp.int32),
            pltpu.VMEM((K, L), jnp.float32),
            pltpu.VMEM((K, L), jnp.float32),
            pltpu.VMEM((K,), jnp.float32),
            pltpu.SemaphoreType.DMA,
            pltpu.SemaphoreType.DMA,
        ],
        compiler_params=pltpu.CompilerParams(
            needs_layout_passes=False, use_tc_tiling_on_sc=False
        ),
    )
    def gather_kernel(tab_hbm, idx_hbm, out_hbm, idx_v, row_v, lane_v,
                      rows_a, rows_b, val_v, sem_a, sem_b):
        wid = lax.axis_index("s") * NC + lax.axis_index("c")
        iota = lax.iota(jnp.int32, L)
        bufs = (rows_a, rows_b)
        sems = (sem_a, sem_b)

        @pl.loop(0, ITERS)
        def _task(i):
            t = wid + i * NW

            @pl.when(t < TASKS)
            def _run():
                g = t % G
                c = t // G
                start = c * K
                pltpu.sync_copy(idx_hbm.at[pl.ds(start, K)], idx_v)

                @pl.loop(0, K // L)
                def _rows(j):
                    iv = idx_v[pl.ds(j * L, L)]
                    row_v[pl.ds(j * L, L)] = (
                        lax.shift_left(lax.shift_right_logical(iv, 7), 6)
                        | (lax.shift_right_logical(iv, 4) & 7)
                    )
                    lane_v[pl.ds(j * L, L)] = iv & (L - 1)

                def row_slice(r):
                    return tab_hbm.at[pl.ds(g * HALF + r * 8, WIN)].at[row_v]

                cp = pltpu.async_copy(row_slice(0), bufs[0], sems[0])
                for r in range(8):
                    cp_next = None
                    if r < 7:
                        cp_next = pltpu.async_copy(
                            row_slice(r + 1), bufs[(r + 1) % 2],
                            sems[(r + 1) % 2],
                        )
                    cp.wait()
                    cp = cp_next
                    buf = bufs[r % 2]

                    @pl.loop(0, K // L)
                    def _extract(j):
                        pos16 = j * L + iota
                        lane16 = lane_v[pl.ds(j * L, L)]
                        val_v[pl.ds(j * L, L)] = plsc.load_gather(
                            buf, [pos16, lane16]
                        )

                    b = 8 * g + r
                    pltpu.sync_copy(
                        val_v, out_hbm.at[pl.ds(b * NSEL_P + start, K)]
                    )

    return gather_kernel


def kernel(params, rp_cat, single_grad, flat_cat):
    B, TOTAL = params.shape
    NSEL = rp_cat.shape[0]
    if TOTAL == NSEL:
        return params
    NSEL_P = ((NSEL + K - 1) // K) * K
    rp = rp_cat.astype(jnp.int32)
    if NSEL_P != NSEL:
        rp = jnp.pad(rp, (0, NSEL_P - NSEL), mode="edge")
    tab = (
        params.reshape(B // 8, 8, TOTAL // 128, 128)
        .transpose(0, 2, 1, 3)
        .reshape(B * TOTAL // L, L)
    )
    out = _build(B, TOTAL, NSEL_P)(tab, rp)
    return out.reshape(B, NSEL_P)[:, :NSEL]


# speedup vs baseline: 2.6434x; 1.1093x over previous
import functools

import jax
import jax.numpy as jnp
from jax import lax
from jax.experimental import pallas as pl
from jax.experimental.pallas import tpu as pltpu
from jax.experimental.pallas import tpu_sc as plsc

NC = 2
NS = 16
NW = NC * NS
K = 2048
L = 16
TK = K // 128


@functools.lru_cache(maxsize=None)
def _build(B, TOTAL, NSEL):
    C = (NSEL + K - 1) // K
    NSEL_P = C * K
    G = B // 8
    TASKS = G * C
    ITERS = (TASKS + NW - 1) // NW
    HALF = TOTAL // 2
    WIN = HALF - 56
    CT = (NSEL + 127) // 128
    LAST_TK = CT - (C - 1) * TK
    mesh = plsc.VectorSubcoreMesh(core_axis_name="c", subcore_axis_name="s")

    @functools.partial(
        pl.kernel,
        out_type=jax.ShapeDtypeStruct((G, CT, 8, 128), jnp.float32),
        mesh=mesh,
        scratch_types=[
            pltpu.VMEM((K,), jnp.int32),
            pltpu.VMEM((K,), jnp.int32),
            pltpu.VMEM((K,), jnp.int32),
            pltpu.VMEM((K, L), jnp.float32),
            pltpu.VMEM((K, L), jnp.float32),
            pltpu.VMEM((TK, 8, 128), jnp.float32),
            pltpu.SemaphoreType.DMA,
            pltpu.SemaphoreType.DMA,
        ],
        compiler_params=pltpu.CompilerParams(
            needs_layout_passes=False, use_tc_tiling_on_sc=False
        ),
    )
    def gather_kernel(tab_hbm, idx_hbm, out_hbm, idx_v, row_v, lane_v,
                      rows_a, rows_b, stage_v, sem_a, sem_b):
        wid = lax.axis_index("s") * NC + lax.axis_index("c")
        bufs = (rows_a, rows_b)
        sems = (sem_a, sem_b)
        iota = lax.iota(jnp.int32, L)

        @pl.loop(0, ITERS)
        def _task(i):
            t = wid + i * NW

            @pl.when(t < TASKS)
            def _run():
                g = t % G
                c = t // G
                start = c * K
                pltpu.sync_copy(idx_hbm.at[pl.ds(start, K)], idx_v)

                @pl.loop(0, K // L)
                def _rows(j):
                    iv = idx_v[pl.ds(j * L, L)]
                    row_v[pl.ds(j * L, L)] = (
                        lax.shift_left(lax.shift_right_logical(iv, 7), 6)
                        | (lax.shift_right_logical(iv, 4) & 7)
                    )
                    lane_v[pl.ds(j * L, L)] = iv & (L - 1)

                def row_slice(r):
                    return tab_hbm.at[pl.ds(g * HALF + r * 8, WIN)].at[row_v]

                cp = pltpu.async_copy(row_slice(0), bufs[0], sems[0])
                for r in range(8):
                    cp_next = None
                    if r < 7:
                        cp_next = pltpu.async_copy(
                            row_slice(r + 1), bufs[(r + 1) % 2],
                            sems[(r + 1) % 2],
                        )
                    cp.wait()
                    cp = cp_next
                    buf = bufs[r % 2]

                    @pl.loop(0, K // L)
                    def _extract(j):
                        pos16 = j * L + iota
                        lane16 = lane_v[pl.ds(j * L, L)]
                        v16 = plsc.load_gather(buf, [pos16, lane16])
                        stage_v[j // 8, r, pl.ds((j % 8) * L, L)] = v16

                @pl.when(c != C - 1)
                def _store():
                    pltpu.sync_copy(
                        stage_v,
                        out_hbm.at[g, pl.ds(c * TK, TK)],
                    )

                @pl.when(c == C - 1)
                def _store_tail():
                    pltpu.sync_copy(
                        stage_v.at[pl.ds(0, LAST_TK)],
                        out_hbm.at[g, pl.ds((C - 1) * TK, LAST_TK)],
                    )

    return gather_kernel, NSEL_P, CT


def kernel(params, rp_cat, single_grad, flat_cat):
    B, TOTAL = params.shape
    NSEL = rp_cat.shape[0]
    if TOTAL == NSEL:
        return params
    gk, NSEL_P, CT = _build(B, TOTAL, NSEL)
    rp = rp_cat.astype(jnp.int32)
    if NSEL_P != NSEL:
        rp = jnp.pad(rp, (0, NSEL_P - NSEL), mode="edge")
    tab = (
        params.reshape(B // 8, 8, TOTAL // 128, 128)
        .transpose(0, 2, 1, 3)
        .reshape(B * TOTAL // L, L)
    )
    out4 = gk(tab, rp)
    out = (
        out4.transpose(0, 2, 1, 3)
        .reshape(B, CT * 128)[:, :NSEL]
    )
    return out
